# baseline (speedup 1.0000x reference)
import jax
import jax.numpy as jnp
from jax import lax
from jax.experimental import pallas as pl
from jax.experimental.pallas import tpu as pltpu

N_EXP = 16
BLOCK_M = 512


def _router_body(w_ref, x_ref, b_ref, w_out_ref, i_out_ref):
    logits = (
        lax.dot_general(
            w_ref[...], x_ref[...],
            dimension_numbers=(((1,), (1,)), ((), ())),
            preferred_element_type=jnp.float32,
        )
        + b_ref[...]
    )
    eidx = jax.lax.broadcasted_iota(jnp.int32, logits.shape, 0)
    neg_inf = jnp.float32(-jnp.inf)

    m1 = jnp.max(logits, axis=0, keepdims=True)
    i1 = jnp.min(jnp.where(logits == m1, eidx, N_EXP), axis=0, keepdims=True)
    masked = jnp.where(eidx == i1, neg_inf, logits)
    m2 = jnp.max(masked, axis=0, keepdims=True)
    i2 = jnp.min(jnp.where(masked == m2, eidx, N_EXP), axis=0, keepdims=True)

    e2 = jnp.exp(m2 - m1)
    denom = e2 + jnp.float32(1.0 + 1e-8)
    w1 = jnp.float32(1.0) / denom
    w2 = e2 / denom
    w_out_ref[...] = jnp.concatenate([w1, w2], axis=0)
    i_out_ref[...] = jnp.concatenate([i1, i2], axis=0)


def kernel(x, gate_weight, expert_bias):
    n_tokens, d_model = x.shape
    bias = expert_bias.reshape(N_EXP, 1)

    w_t, i_t = pl.pallas_call(
        _router_body,
        grid=(n_tokens // BLOCK_M,),
        in_specs=[
            pl.BlockSpec((N_EXP, d_model), lambda i: (0, 0)),
            pl.BlockSpec((BLOCK_M, d_model), lambda i: (i, 0)),
            pl.BlockSpec((N_EXP, 1), lambda i: (0, 0)),
        ],
        out_specs=[
            pl.BlockSpec((2, BLOCK_M), lambda i: (0, i)),
            pl.BlockSpec((2, BLOCK_M), lambda i: (0, i)),
        ],
        out_shape=[
            jax.ShapeDtypeStruct((2, n_tokens), jnp.float32),
            jax.ShapeDtypeStruct((2, n_tokens), jnp.int32),
        ],
        compiler_params=pltpu.CompilerParams(
            dimension_semantics=("arbitrary",),
        ),
    )(gate_weight, x, bias)
    return (w_t.T, i_t.T)


# speedup vs baseline: 1.2058x; 1.2058x over previous
import jax
import jax.numpy as jnp
from jax import lax
from jax.experimental import pallas as pl
from jax.experimental.pallas import tpu as pltpu

N_EXP = 16
BLOCK_M = 1024


def _router_body(w_ref, x_ref, b_ref, w_out_ref, i_out_ref):
    logits = (
        lax.dot_general(
            w_ref[...], x_ref[...],
            dimension_numbers=(((1,), (1,)), ((), ())),
            preferred_element_type=jnp.float32,
        )
        + b_ref[...]
    )
    eidx = jax.lax.broadcasted_iota(jnp.int32, logits.shape, 0)
    neg_inf = jnp.float32(-jnp.inf)

    m1 = jnp.max(logits, axis=0, keepdims=True)
    i1 = jnp.min(jnp.where(logits == m1, eidx, N_EXP), axis=0, keepdims=True)
    masked = jnp.where(eidx == i1, neg_inf, logits)
    m2 = jnp.max(masked, axis=0, keepdims=True)
    i2 = jnp.min(jnp.where(masked == m2, eidx, N_EXP), axis=0, keepdims=True)

    e2 = jnp.exp(m2 - m1)
    denom = e2 + jnp.float32(1.0 + 1e-8)
    w1 = jnp.float32(1.0) / denom
    w2 = e2 / denom
    w_out_ref[...] = jnp.concatenate([w1, w2], axis=0)
    i_out_ref[...] = jnp.concatenate([i1, i2], axis=0)


def kernel(x, gate_weight, expert_bias):
    n_tokens, d_model = x.shape
    bias = expert_bias.reshape(N_EXP, 1)

    w_t, i_t = pl.pallas_call(
        _router_body,
        grid=(n_tokens // BLOCK_M,),
        in_specs=[
            pl.BlockSpec((N_EXP, d_model), lambda i: (0, 0)),
            pl.BlockSpec((BLOCK_M, d_model), lambda i: (i, 0)),
            pl.BlockSpec((N_EXP, 1), lambda i: (0, 0)),
        ],
        out_specs=[
            pl.BlockSpec((2, BLOCK_M), lambda i: (0, i)),
            pl.BlockSpec((2, BLOCK_M), lambda i: (0, i)),
        ],
        out_shape=[
            jax.ShapeDtypeStruct((2, n_tokens), jnp.float32),
            jax.ShapeDtypeStruct((2, n_tokens), jnp.int32),
        ],
        compiler_params=pltpu.CompilerParams(
            dimension_semantics=("parallel",),
        ),
    )(gate_weight, x, bias)
    return (w_t.T, i_t.T)
